# Initial kernel scaffold; baseline (speedup 1.0000x reference)
import jax
import jax.numpy as jnp
from jax.experimental import pallas as pl


def kernel(grid_obs, state_table, pos_table):
    raise NotImplementedError("write your pallas kernel here")


# speedup vs baseline: 8.4404x; 8.4404x over previous
import functools

import jax
import jax.numpy as jnp
from jax import lax
from jax.experimental import pallas as pl
from jax.experimental.pallas import tpu as pltpu
from jax.experimental.pallas import tpu_sc as plsc

NUM_STATES = 12
EMBED_DIM = 64
N_POS = 1024
BATCH = 4096

NC = 2
NS = 16
L = 16
NW = NC * NS
B_PER_W = BATCH // NW
CHUNK = 128
N_CHUNKS = N_POS // CHUNK
POS_PER_TILE = N_POS // NS
C_ROWS = N_POS * NUM_STATES

_mesh = plsc.VectorSubcoreMesh(core_axis_name="c", subcore_axis_name="s")


@functools.partial(
    pl.kernel,
    out_type=[
        jax.ShapeDtypeStruct((BATCH, N_POS, EMBED_DIM), jnp.float32),
        jax.ShapeDtypeStruct((NC * C_ROWS, EMBED_DIM), jnp.float32),
    ],
    mesh=_mesh,
    compiler_params=pltpu.CompilerParams(use_tc_tiling_on_sc=False),
    scratch_types=[
        pltpu.VMEM((NUM_STATES, EMBED_DIM), jnp.float32),
        pltpu.VMEM((POS_PER_TILE, EMBED_DIM), jnp.float32),
        pltpu.VMEM((POS_PER_TILE * NUM_STATES, EMBED_DIM), jnp.float32),
        pltpu.VMEM((N_POS,), jnp.int32),
        pltpu.VMEM((N_POS,), jnp.int32),
        pltpu.VMEM((N_POS, EMBED_DIM), jnp.float32),
        pltpu.SemaphoreType.DMA,
    ],
)
def _sc_embed(grid_hbm, state_hbm, pos_hbm, out_hbm, c_hbm,
              sbuf, pbuf, cbuf, pvec, ibuf, rbuf, sem):
    cid = lax.axis_index("c")
    sid = lax.axis_index("s")
    wid = sid * NC + cid

    pltpu.sync_copy(state_hbm, sbuf)
    pltpu.sync_copy(pos_hbm.at[pl.ds(sid * POS_PER_TILE, POS_PER_TILE)], pbuf)

    def build_row(i, carry):
        for s in range(NUM_STATES):
            for d in range(EMBED_DIM // L):
                cbuf[i * NUM_STATES + s, pl.ds(d * L, L)] = (
                    pbuf[i, pl.ds(d * L, L)] + sbuf[s, pl.ds(d * L, L)]
                )
        return carry

    lax.fori_loop(0, POS_PER_TILE, build_row, 0)
    pltpu.sync_copy(
        cbuf,
        c_hbm.at[pl.ds(cid * C_ROWS + sid * POS_PER_TILE * NUM_STATES,
                       POS_PER_TILE * NUM_STATES)],
    )

    def pvec_body(k, carry):
        pvec[pl.ds(k * L, L)] = (
            lax.iota(jnp.int32, L) * NUM_STATES
            + (k * (L * NUM_STATES) + cid * C_ROWS)
        )
        return carry

    lax.fori_loop(0, N_POS // L, pvec_body, 0)

    plsc.subcore_barrier()

    def batch_body(i, carry):
        b = wid * B_PER_W + i
        pltpu.sync_copy(grid_hbm.at[b], ibuf)

        def off_body(k, c2):
            ibuf[pl.ds(k * L, L)] = ibuf[pl.ds(k * L, L)] + pvec[pl.ds(k * L, L)]
            return c2

        lax.fori_loop(0, N_POS // L, off_body, 0)

        copies = [
            pltpu.async_copy(
                c_hbm.at[ibuf.at[pl.ds(j * CHUNK, CHUNK)]],
                rbuf.at[pl.ds(j * CHUNK, CHUNK)],
                sem,
            )
            for j in range(N_CHUNKS)
        ]
        for c in copies:
            c.wait()
        pltpu.sync_copy(rbuf, out_hbm.at[b])
        return carry

    lax.fori_loop(0, B_PER_W, batch_body, 0)


def kernel(grid_obs, state_table, pos_table):
    out, _ = _sc_embed(grid_obs.astype(jnp.int32), state_table, pos_table)
    return out.reshape(BATCH, N_POS * EMBED_DIM)


# speedup vs baseline: 8.7780x; 1.0400x over previous
import functools

import jax
import jax.numpy as jnp
from jax import lax
from jax.experimental import pallas as pl
from jax.experimental.pallas import tpu as pltpu
from jax.experimental.pallas import tpu_sc as plsc

NUM_STATES = 12
EMBED_DIM = 64
N_POS = 1024
BATCH = 4096

NC = 2
NS = 16
L = 16
NW = NC * NS
B_PER_W = BATCH // NW
CHUNK = 128
HALF = 512
POS_PER_TILE = N_POS // NS
C_ROWS = N_POS * NUM_STATES

_mesh = plsc.VectorSubcoreMesh(core_axis_name="c", subcore_axis_name="s")


@functools.partial(
    pl.kernel,
    out_type=[
        jax.ShapeDtypeStruct((BATCH, N_POS, EMBED_DIM), jnp.float32),
        jax.ShapeDtypeStruct((NC * C_ROWS, EMBED_DIM), jnp.float32),
    ],
    mesh=_mesh,
    compiler_params=pltpu.CompilerParams(use_tc_tiling_on_sc=False),
    scratch_types=[
        pltpu.VMEM((NUM_STATES, EMBED_DIM), jnp.float32),
        pltpu.VMEM((POS_PER_TILE, EMBED_DIM), jnp.float32),
        pltpu.VMEM((POS_PER_TILE * NUM_STATES, EMBED_DIM), jnp.float32),
        pltpu.VMEM((N_POS,), jnp.int32),
        pltpu.VMEM((2, N_POS), jnp.int32),
        pltpu.VMEM((2, HALF, EMBED_DIM), jnp.float32),
        pltpu.SemaphoreType.DMA,
        pltpu.SemaphoreType.DMA,
        pltpu.SemaphoreType.DMA,
        pltpu.SemaphoreType.DMA,
    ],
)
def _sc_embed(grid_hbm, state_hbm, pos_hbm, out_hbm, c_hbm,
              sbuf, pbuf, cbuf, pvec, ibuf, rbuf, gsem, isem, wsem0, wsem1):
    cid = lax.axis_index("c")
    sid = lax.axis_index("s")
    wid = sid * NC + cid

    pltpu.sync_copy(state_hbm, sbuf)
    pltpu.sync_copy(pos_hbm.at[pl.ds(sid * POS_PER_TILE, POS_PER_TILE)], pbuf)

    def build_row(i, carry):
        for s in range(NUM_STATES):
            for d in range(EMBED_DIM // L):
                cbuf[i * NUM_STATES + s, pl.ds(d * L, L)] = (
                    pbuf[i, pl.ds(d * L, L)] + sbuf[s, pl.ds(d * L, L)]
                )
        return carry

    lax.fori_loop(0, POS_PER_TILE, build_row, 0)
    pltpu.sync_copy(
        cbuf,
        c_hbm.at[pl.ds(cid * C_ROWS + sid * POS_PER_TILE * NUM_STATES,
                       POS_PER_TILE * NUM_STATES)],
    )

    def pvec_body(k, carry):
        pvec[pl.ds(k * L, L)] = (
            lax.iota(jnp.int32, L) * NUM_STATES
            + (k * (L * NUM_STATES) + cid * C_ROWS)
        )
        return carry

    lax.fori_loop(0, N_POS // L, pvec_body, 0)

    plsc.subcore_barrier()

    pltpu.async_copy(grid_hbm.at[wid * B_PER_W], ibuf.at[0], isem)

    def batch_body(i, carry):
        b = wid * B_PER_W + i
        sl = lax.rem(i, 2)
        pltpu.make_async_copy(grid_hbm.at[b], ibuf.at[sl], isem).wait()

        @pl.when(i < B_PER_W - 1)
        def _prefetch_idx():
            pltpu.async_copy(grid_hbm.at[b + 1], ibuf.at[1 - sl], isem)

        def off_body(k, c2):
            for j in range(4):
                ibuf[sl, pl.ds((k * 4 + j) * L, L)] = (
                    ibuf[sl, pl.ds((k * 4 + j) * L, L)]
                    + pvec[pl.ds((k * 4 + j) * L, L)]
                )
            return c2

        lax.fori_loop(0, N_POS // (4 * L), off_body, 0)

        for h in range(2):
            wsem = wsem0 if h == 0 else wsem1

            @pl.when(i > 0)
            def _wait_prev_write():
                pltpu.make_async_copy(
                    rbuf.at[h], out_hbm.at[b, pl.ds(h * HALF, HALF)], wsem
                ).wait()

            gathers = [
                pltpu.async_copy(
                    c_hbm.at[ibuf.at[sl, pl.ds(h * HALF + j * CHUNK, CHUNK)]],
                    rbuf.at[h, pl.ds(j * CHUNK, CHUNK)],
                    gsem,
                )
                for j in range(HALF // CHUNK)
            ]
            for g in gathers:
                g.wait()
            pltpu.async_copy(rbuf.at[h], out_hbm.at[b, pl.ds(h * HALF, HALF)], wsem)
        return carry

    lax.fori_loop(0, B_PER_W, batch_body, 0)

    last = wid * B_PER_W + B_PER_W - 1
    for h in range(2):
        wsem = wsem0 if h == 0 else wsem1
        pltpu.make_async_copy(
            rbuf.at[h], out_hbm.at[last, pl.ds(h * HALF, HALF)], wsem
        ).wait()


def kernel(grid_obs, state_table, pos_table):
    out, _ = _sc_embed(grid_obs.astype(jnp.int32), state_table, pos_table)
    return out.reshape(BATCH, N_POS * EMBED_DIM)


# speedup vs baseline: 20.2887x; 2.3113x over previous
import functools

import jax
import jax.numpy as jnp
from jax import lax
from jax.experimental import pallas as pl
from jax.experimental.pallas import tpu as pltpu
from jax.experimental.pallas import tpu_sc as plsc

NUM_STATES = 12
NPAIR_STATES = NUM_STATES * NUM_STATES
EMBED_DIM = 64
ROW = 2 * EMBED_DIM
N_POS = 1024
N_PP = N_POS // 2
BATCH = 4096

NC = 2
NS = 16
L = 16
NW = NC * NS
B_PER_W = BATCH // NW
CHUNK = 128
HALF = 256
T_ROWS = N_PP * NPAIR_STATES
PP_PER_TILE = N_PP // NS

_mesh = plsc.VectorSubcoreMesh(core_axis_name="c", subcore_axis_name="s")


@functools.partial(
    pl.kernel,
    out_type=[
        jax.ShapeDtypeStruct((BATCH, N_PP, ROW), jnp.float32),
        jax.ShapeDtypeStruct((NC * T_ROWS, ROW), jnp.float32),
    ],
    mesh=_mesh,
    compiler_params=pltpu.CompilerParams(
        use_tc_tiling_on_sc=False, needs_layout_passes=False
    ),
    scratch_types=[
        pltpu.VMEM((NUM_STATES, EMBED_DIM), jnp.float32),
        pltpu.VMEM((2 * PP_PER_TILE, EMBED_DIM), jnp.float32),
        pltpu.VMEM((NPAIR_STATES, ROW), jnp.float32),
        pltpu.VMEM((N_PP,), jnp.int32),
        pltpu.VMEM((2, N_POS), jnp.int32),
        pltpu.VMEM((N_PP,), jnp.int32),
        pltpu.VMEM((2, HALF, ROW), jnp.float32),
        pltpu.SemaphoreType.DMA,
        pltpu.SemaphoreType.DMA,
        pltpu.SemaphoreType.DMA,
        pltpu.SemaphoreType.DMA,
    ],
)
def _sc_embed(grid_hbm, state_hbm, pos_hbm, out_hbm, t_hbm,
              sbuf, pbuf, bbuf, pvec, ibuf, pibuf, rbuf,
              gsem, isem, wsem0, wsem1):
    cid = lax.axis_index("c")
    sid = lax.axis_index("s")
    wid = sid * NC + cid

    pltpu.sync_copy(state_hbm, sbuf)
    pltpu.sync_copy(pos_hbm.at[pl.ds(sid * 2 * PP_PER_TILE, 2 * PP_PER_TILE)], pbuf)

    def build_pair(ppl, carry):
        def combo(r, c2):
            se = lax.div(r, NUM_STATES)
            so = lax.rem(r, NUM_STATES)
            for d in range(EMBED_DIM // L):
                bbuf[r, pl.ds(d * L, L)] = (
                    pbuf[2 * ppl, pl.ds(d * L, L)] + sbuf[se, pl.ds(d * L, L)]
                )
                bbuf[r, pl.ds(EMBED_DIM + d * L, L)] = (
                    pbuf[2 * ppl + 1, pl.ds(d * L, L)] + sbuf[so, pl.ds(d * L, L)]
                )
            return c2

        lax.fori_loop(0, NPAIR_STATES, combo, 0)
        pltpu.sync_copy(
            bbuf,
            t_hbm.at[pl.ds(cid * T_ROWS + (sid * PP_PER_TILE + ppl) * NPAIR_STATES,
                           NPAIR_STATES)],
        )
        return carry

    lax.fori_loop(0, PP_PER_TILE, build_pair, 0)

    def pvec_body(k, carry):
        pvec[pl.ds(k * L, L)] = (
            lax.iota(jnp.int32, L) * NPAIR_STATES
            + (k * (L * NPAIR_STATES) + cid * T_ROWS)
        )
        return carry

    lax.fori_loop(0, N_PP // L, pvec_body, 0)

    plsc.subcore_barrier()

    pltpu.async_copy(grid_hbm.at[wid * B_PER_W], ibuf.at[0], isem)

    def batch_body(i, carry):
        b = wid * B_PER_W + i
        sl = lax.rem(i, 2)
        pltpu.make_async_copy(grid_hbm.at[b], ibuf.at[sl], isem).wait()

        @pl.when(i < B_PER_W - 1)
        def _prefetch_idx():
            pltpu.async_copy(grid_hbm.at[b + 1], ibuf.at[1 - sl], isem)

        slv = jnp.full((L,), sl, dtype=jnp.int32)

        def pidx_body(k, c2):
            cols = lax.iota(jnp.int32, L) * 2 + k * (2 * L)
            even = plsc.load_gather(ibuf, [slv, cols])
            odd = plsc.load_gather(ibuf, [slv, cols + 1])
            pibuf[pl.ds(k * L, L)] = (
                even * NUM_STATES + odd + pvec[pl.ds(k * L, L)]
            )
            return c2

        lax.fori_loop(0, N_PP // L, pidx_body, 0)

        for h in range(2):
            wsem = wsem0 if h == 0 else wsem1

            @pl.when(i > 0)
            def _wait_prev_write():
                pltpu.make_async_copy(
                    rbuf.at[h], out_hbm.at[b, pl.ds(h * HALF, HALF)], wsem
                ).wait()

            gathers = [
                pltpu.async_copy(
                    t_hbm.at[pibuf.at[pl.ds(h * HALF + j * CHUNK, CHUNK)]],
                    rbuf.at[h, pl.ds(j * CHUNK, CHUNK)],
                    gsem,
                )
                for j in range(HALF // CHUNK)
            ]
            for g in gathers:
                g.wait()
            pltpu.async_copy(rbuf.at[h], out_hbm.at[b, pl.ds(h * HALF, HALF)], wsem)
        return carry

    lax.fori_loop(0, B_PER_W, batch_body, 0)

    last = wid * B_PER_W + B_PER_W - 1
    for h in range(2):
        wsem = wsem0 if h == 0 else wsem1
        pltpu.make_async_copy(
            rbuf.at[h], out_hbm.at[last, pl.ds(h * HALF, HALF)], wsem
        ).wait()


def kernel(grid_obs, state_table, pos_table):
    out, _ = _sc_embed(grid_obs.astype(jnp.int32), state_table, pos_table)
    return out.reshape(BATCH, N_POS * EMBED_DIM)


# speedup vs baseline: 21.8567x; 1.0773x over previous
import functools

import jax
import jax.numpy as jnp
from jax import lax
from jax.experimental import pallas as pl
from jax.experimental.pallas import tpu as pltpu
from jax.experimental.pallas import tpu_sc as plsc

NUM_STATES = 12
NPAIR_STATES = NUM_STATES * NUM_STATES
EMBED_DIM = 64
ROW = 2 * EMBED_DIM
N_POS = 1024
N_PP = N_POS // 2
BATCH = 4096

NC = 2
NS = 16
L = 16
NW = NC * NS
B_PER_W = BATCH // NW
CHUNK = 128
HALF = 256
T_ROWS = N_PP * NPAIR_STATES
PP_PER_TILE = N_PP // NS

_mesh = plsc.VectorSubcoreMesh(core_axis_name="c", subcore_axis_name="s")


@functools.partial(
    pl.kernel,
    out_type=[
        jax.ShapeDtypeStruct((BATCH, N_PP, ROW), jnp.float32),
        jax.ShapeDtypeStruct((NC * T_ROWS, ROW), jnp.float32),
    ],
    mesh=_mesh,
    compiler_params=pltpu.CompilerParams(
        use_tc_tiling_on_sc=False, needs_layout_passes=False
    ),
    scratch_types=[
        pltpu.VMEM((NUM_STATES, EMBED_DIM), jnp.float32),
        pltpu.VMEM((2 * PP_PER_TILE, EMBED_DIM), jnp.float32),
        pltpu.VMEM((2, NPAIR_STATES, ROW), jnp.float32),
        pltpu.VMEM((2 * NUM_STATES, EMBED_DIM), jnp.float32),
        pltpu.VMEM((N_PP,), jnp.int32),
        pltpu.VMEM((2, N_POS), jnp.int32),
        pltpu.VMEM((N_PP,), jnp.int32),
        pltpu.VMEM((2, HALF, ROW), jnp.float32),
        pltpu.SemaphoreType.DMA,
        pltpu.SemaphoreType.DMA,
        pltpu.SemaphoreType.DMA,
        pltpu.SemaphoreType.DMA,
    ],
)
def _sc_embed(grid_hbm, state_hbm, pos_hbm, out_hbm, t_hbm,
              sbuf, pbuf, bbuf, lrtmp, pvec, ibuf, pibuf, rbuf,
              gsem, isem, wsem0, wsem1):
    cid = lax.axis_index("c")
    sid = lax.axis_index("s")
    wid = sid * NC + cid

    pltpu.sync_copy(state_hbm, sbuf)
    pltpu.sync_copy(pos_hbm.at[pl.ds(sid * 2 * PP_PER_TILE, 2 * PP_PER_TILE)], pbuf)

    def build_pair2(pp2, carry):
        for q in range(2):
            ppl = pp2 * 2 + q
            bwsem = wsem0 if q == 0 else wsem1

            def halves(s, c2):
                for d in range(EMBED_DIM // L):
                    lrtmp[s, pl.ds(d * L, L)] = (
                        pbuf[2 * ppl, pl.ds(d * L, L)] + sbuf[s, pl.ds(d * L, L)]
                    )
                    lrtmp[NUM_STATES + s, pl.ds(d * L, L)] = (
                        pbuf[2 * ppl + 1, pl.ds(d * L, L)] + sbuf[s, pl.ds(d * L, L)]
                    )
                return c2

            lax.fori_loop(0, NUM_STATES, halves, 0)

            @pl.when(pp2 >= 1)
            def _wait_prev_slab():
                pltpu.make_async_copy(
                    bbuf.at[q], t_hbm.at[pl.ds(0, NPAIR_STATES)], bwsem
                ).wait()

            def se_body(se, c2):
                lvals = [lrtmp[se, pl.ds(d * L, L)] for d in range(EMBED_DIM // L)]

                def so_body(so, lv):
                    r = se * NUM_STATES + so
                    for d in range(EMBED_DIM // L):
                        bbuf[q, r, pl.ds(d * L, L)] = lv[d]
                        bbuf[q, r, pl.ds(EMBED_DIM + d * L, L)] = (
                            lrtmp[NUM_STATES + so, pl.ds(d * L, L)]
                        )
                    return lv

                lax.fori_loop(0, NUM_STATES, so_body, lvals)
                return c2

            lax.fori_loop(0, NUM_STATES, se_body, 0)
            pltpu.async_copy(
                bbuf.at[q],
                t_hbm.at[pl.ds(
                    cid * T_ROWS + (sid * PP_PER_TILE + ppl) * NPAIR_STATES,
                    NPAIR_STATES)],
                bwsem,
            )
        return carry

    lax.fori_loop(0, PP_PER_TILE // 2, build_pair2, 0)
    for q in range(2):
        bwsem = wsem0 if q == 0 else wsem1
        pltpu.make_async_copy(
            bbuf.at[q], t_hbm.at[pl.ds(0, NPAIR_STATES)], bwsem
        ).wait()

    def pvec_body(k, carry):
        pvec[pl.ds(k * L, L)] = (
            lax.iota(jnp.int32, L) * NPAIR_STATES
            + (k * (L * NPAIR_STATES) + cid * T_ROWS)
        )
        return carry

    lax.fori_loop(0, N_PP // L, pvec_body, 0)

    plsc.subcore_barrier()

    pltpu.async_copy(grid_hbm.at[wid * B_PER_W], ibuf.at[0], isem)

    def batch_body(i, carry):
        b = wid * B_PER_W + i
        sl = lax.rem(i, 2)
        pltpu.make_async_copy(grid_hbm.at[b], ibuf.at[sl], isem).wait()

        @pl.when(i < B_PER_W - 1)
        def _prefetch_idx():
            pltpu.async_copy(grid_hbm.at[b + 1], ibuf.at[1 - sl], isem)

        slv = jnp.full((L,), sl, dtype=jnp.int32)

        def pidx_body(k, c2):
            cols = lax.iota(jnp.int32, L) * 2 + k * (2 * L)
            even = plsc.load_gather(ibuf, [slv, cols])
            odd = plsc.load_gather(ibuf, [slv, cols + 1])
            pibuf[pl.ds(k * L, L)] = (
                even * NUM_STATES + odd + pvec[pl.ds(k * L, L)]
            )
            return c2

        lax.fori_loop(0, N_PP // L, pidx_body, 0)

        for h in range(2):
            wsem = wsem0 if h == 0 else wsem1

            @pl.when(i > 0)
            def _wait_prev_write():
                pltpu.make_async_copy(
                    rbuf.at[h], out_hbm.at[b, pl.ds(h * HALF, HALF)], wsem
                ).wait()

            gathers = [
                pltpu.async_copy(
                    t_hbm.at[pibuf.at[pl.ds(h * HALF + j * CHUNK, CHUNK)]],
                    rbuf.at[h, pl.ds(j * CHUNK, CHUNK)],
                    gsem,
                )
                for j in range(HALF // CHUNK)
            ]
            for g in gathers:
                g.wait()
            pltpu.async_copy(rbuf.at[h], out_hbm.at[b, pl.ds(h * HALF, HALF)], wsem)
        return carry

    lax.fori_loop(0, B_PER_W, batch_body, 0)

    last = wid * B_PER_W + B_PER_W - 1
    for h in range(2):
        wsem = wsem0 if h == 0 else wsem1
        pltpu.make_async_copy(
            rbuf.at[h], out_hbm.at[last, pl.ds(h * HALF, HALF)], wsem
        ).wait()


def kernel(grid_obs, state_table, pos_table):
    out, _ = _sc_embed(grid_obs.astype(jnp.int32), state_table, pos_table)
    return out.reshape(BATCH, N_POS * EMBED_DIM)


# speedup vs baseline: 21.8719x; 1.0007x over previous
import functools

import jax
import jax.numpy as jnp
from jax import lax
from jax.experimental import pallas as pl
from jax.experimental.pallas import tpu as pltpu
from jax.experimental.pallas import tpu_sc as plsc

NUM_STATES = 12
NPAIR_STATES = NUM_STATES * NUM_STATES
EMBED_DIM = 64
ROW = 2 * EMBED_DIM
N_POS = 1024
N_PP = N_POS // 2
BATCH = 4096

NC = 2
NS = 16
L = 16
NW = NC * NS
B_PER_W = BATCH // NW
CHUNK = 128
HALF = 256
T_ROWS = N_PP * NPAIR_STATES
PP_PER_TILE = N_PP // NS

_mesh = plsc.VectorSubcoreMesh(core_axis_name="c", subcore_axis_name="s")


@functools.partial(
    pl.kernel,
    out_type=[
        jax.ShapeDtypeStruct((BATCH, N_PP, ROW), jnp.float32),
        jax.ShapeDtypeStruct((NC * T_ROWS, ROW), jnp.float32),
    ],
    mesh=_mesh,
    compiler_params=pltpu.CompilerParams(
        use_tc_tiling_on_sc=False, needs_layout_passes=False
    ),
    scratch_types=[
        pltpu.VMEM((NUM_STATES, EMBED_DIM), jnp.float32),
        pltpu.VMEM((2 * PP_PER_TILE, EMBED_DIM), jnp.float32),
        pltpu.VMEM((2, NPAIR_STATES, ROW), jnp.float32),
        pltpu.VMEM((2 * NUM_STATES, EMBED_DIM), jnp.float32),
        pltpu.VMEM((N_PP,), jnp.int32),
        pltpu.VMEM((2, N_POS), jnp.int32),
        pltpu.VMEM((2, N_PP), jnp.int32),
        pltpu.VMEM((2, HALF, ROW), jnp.float32),
        pltpu.SemaphoreType.DMA,
        pltpu.SemaphoreType.DMA,
        pltpu.SemaphoreType.DMA,
        pltpu.SemaphoreType.DMA,
        pltpu.SemaphoreType.DMA,
    ],
)
def _sc_embed(grid_hbm, state_hbm, pos_hbm, out_hbm, t_hbm,
              sbuf, pbuf, bbuf, lrtmp, pvec, ibuf, pibuf, rbuf,
              gsem0, gsem1, isem, wsem0, wsem1):
    cid = lax.axis_index("c")
    sid = lax.axis_index("s")
    wid = sid * NC + cid

    pltpu.sync_copy(state_hbm, sbuf)
    pltpu.sync_copy(pos_hbm.at[pl.ds(sid * 2 * PP_PER_TILE, 2 * PP_PER_TILE)], pbuf)

    def build_pair2(pp2, carry):
        for q in range(2):
            ppl = pp2 * 2 + q
            bwsem = wsem0 if q == 0 else wsem1

            def halves(s, c2):
                for d in range(EMBED_DIM // L):
                    lrtmp[s, pl.ds(d * L, L)] = (
                        pbuf[2 * ppl, pl.ds(d * L, L)] + sbuf[s, pl.ds(d * L, L)]
                    )
                    lrtmp[NUM_STATES + s, pl.ds(d * L, L)] = (
                        pbuf[2 * ppl + 1, pl.ds(d * L, L)] + sbuf[s, pl.ds(d * L, L)]
                    )
                return c2

            lax.fori_loop(0, NUM_STATES, halves, 0)

            @pl.when(pp2 >= 1)
            def _wait_prev_slab():
                pltpu.make_async_copy(
                    bbuf.at[q], t_hbm.at[pl.ds(0, NPAIR_STATES)], bwsem
                ).wait()

            def se_body(se, c2):
                lvals = [lrtmp[se, pl.ds(d * L, L)] for d in range(EMBED_DIM // L)]

                def so_body(so, lv):
                    r = se * NUM_STATES + so
                    for d in range(EMBED_DIM // L):
                        bbuf[q, r, pl.ds(d * L, L)] = lv[d]
                        bbuf[q, r, pl.ds(EMBED_DIM + d * L, L)] = (
                            lrtmp[NUM_STATES + so, pl.ds(d * L, L)]
                        )
                    return lv

                lax.fori_loop(0, NUM_STATES, so_body, lvals)
                return c2

            lax.fori_loop(0, NUM_STATES, se_body, 0)
            pltpu.async_copy(
                bbuf.at[q],
                t_hbm.at[pl.ds(
                    cid * T_ROWS + (sid * PP_PER_TILE + ppl) * NPAIR_STATES,
                    NPAIR_STATES)],
                bwsem,
            )
        return carry

    lax.fori_loop(0, PP_PER_TILE // 2, build_pair2, 0)
    for q in range(2):
        bwsem = wsem0 if q == 0 else wsem1
        pltpu.make_async_copy(
            bbuf.at[q], t_hbm.at[pl.ds(0, NPAIR_STATES)], bwsem
        ).wait()

    def pvec_body(k, carry):
        pvec[pl.ds(k * L, L)] = (
            lax.iota(jnp.int32, L) * NPAIR_STATES
            + (k * (L * NPAIR_STATES) + cid * T_ROWS)
        )
        return carry

    lax.fori_loop(0, N_PP // L, pvec_body, 0)

    plsc.subcore_barrier()


    def make_pidx(sl, c2):
        slv = jnp.full((L,), sl, dtype=jnp.int32)

        def pidx_body(k, c3):
            cols = lax.iota(jnp.int32, L) * 2 + k * (2 * L)
            even = plsc.load_gather(ibuf, [slv, cols])
            odd = plsc.load_gather(ibuf, [slv, cols + 1])
            pibuf[sl, pl.ds(k * L, L)] = (
                even * NUM_STATES + odd + pvec[pl.ds(k * L, L)]
            )
            return c3

        return lax.fori_loop(0, N_PP // L, pidx_body, c2)

    b0 = wid * B_PER_W
    pltpu.sync_copy(grid_hbm.at[b0], ibuf.at[0])
    make_pidx(0, 0)
    pltpu.async_copy(grid_hbm.at[b0 + 1], ibuf.at[1], isem)

    def batch_body(i, carry):
        b = b0 + i
        sl = lax.rem(i, 2)

        @pl.when(i > 0)
        def _wait_w0():
            pltpu.make_async_copy(
                rbuf.at[0], out_hbm.at[b, pl.ds(0, HALF)], wsem0
            ).wait()

        g0 = [
            pltpu.async_copy(
                t_hbm.at[pibuf.at[sl, pl.ds(j * CHUNK, CHUNK)]],
                rbuf.at[0, pl.ds(j * CHUNK, CHUNK)],
                gsem0,
            )
            for j in range(HALF // CHUNK)
        ]

        @pl.when(i > 0)
        def _wait_w1():
            pltpu.make_async_copy(
                rbuf.at[1], out_hbm.at[b, pl.ds(HALF, HALF)], wsem1
            ).wait()

        g1 = [
            pltpu.async_copy(
                t_hbm.at[pibuf.at[sl, pl.ds(HALF + j * CHUNK, CHUNK)]],
                rbuf.at[1, pl.ds(j * CHUNK, CHUNK)],
                gsem1,
            )
            for j in range(HALF // CHUNK)
        ]

        @pl.when(i < B_PER_W - 1)
        def _next_pidx():
            pltpu.make_async_copy(grid_hbm.at[b + 1], ibuf.at[1 - sl], isem).wait()

            @pl.when(i < B_PER_W - 2)
            def _stage_next_idx():
                pltpu.async_copy(grid_hbm.at[b + 2], ibuf.at[sl], isem)

            make_pidx(1 - sl, 0)

        for g in g0:
            g.wait()
        pltpu.async_copy(rbuf.at[0], out_hbm.at[b, pl.ds(0, HALF)], wsem0)
        for g in g1:
            g.wait()
        pltpu.async_copy(rbuf.at[1], out_hbm.at[b, pl.ds(HALF, HALF)], wsem1)
        return carry

    lax.fori_loop(0, B_PER_W, batch_body, 0)

    last = wid * B_PER_W + B_PER_W - 1
    for h in range(2):
        wsem = wsem0 if h == 0 else wsem1
        pltpu.make_async_copy(
            rbuf.at[h], out_hbm.at[last, pl.ds(h * HALF, HALF)], wsem
        ).wait()


def kernel(grid_obs, state_table, pos_table):
    out, _ = _sc_embed(grid_obs.astype(jnp.int32), state_table, pos_table)
    return out.reshape(BATCH, N_POS * EMBED_DIM)


# speedup vs baseline: 39.0238x; 1.7842x over previous
import functools

import jax
import jax.numpy as jnp
from jax import lax
from jax.experimental import pallas as pl
from jax.experimental.pallas import tpu as pltpu
from jax.experimental.pallas import tpu_sc as plsc

NUM_STATES = 12
NPAIR_STATES = NUM_STATES * NUM_STATES
EMBED_DIM = 64
ROW = 2 * EMBED_DIM
N_POS = 1024
N_PP = N_POS // 2
BATCH = 4096
GRP = 8
N_GRP = BATCH // GRP

NC = 2
NS = 16
L = 16
NW = NC * NS
G_PER_W = N_GRP // NW
CHUNK = 128
UNIT = 256
N_UNITS = (N_PP * GRP) // UNIT
T_ROWS = N_PP * NPAIR_STATES
PP_PER_TILE = N_PP // NS

_mesh = plsc.VectorSubcoreMesh(core_axis_name="c", subcore_axis_name="s")


@functools.partial(
    pl.kernel,
    out_type=[
        jax.ShapeDtypeStruct((N_GRP, N_PP * GRP, ROW), jnp.float32),
        jax.ShapeDtypeStruct((NC * T_ROWS, ROW), jnp.float32),
    ],
    mesh=_mesh,
    compiler_params=pltpu.CompilerParams(
        use_tc_tiling_on_sc=False, needs_layout_passes=False
    ),
    scratch_types=[
        pltpu.VMEM((NUM_STATES, EMBED_DIM), jnp.float32),
        pltpu.VMEM((2 * PP_PER_TILE, EMBED_DIM), jnp.float32),
        pltpu.VMEM((2, NPAIR_STATES, ROW), jnp.float32),
        pltpu.VMEM((2 * NUM_STATES, EMBED_DIM), jnp.float32),
        pltpu.VMEM((N_PP * GRP,), jnp.int32),
        pltpu.VMEM((GRP, N_POS), jnp.int32),
        pltpu.VMEM((N_PP * GRP,), jnp.int32),
        pltpu.VMEM((2, UNIT, ROW), jnp.float32),
        pltpu.SemaphoreType.DMA,
        pltpu.SemaphoreType.DMA,
        pltpu.SemaphoreType.DMA,
        pltpu.SemaphoreType.DMA,
    ],
)
def _sc_embed(grid_hbm, state_hbm, pos_hbm, out_hbm, t_hbm,
              sbuf, pbuf, bbuf, lrtmp, pvec, ibuf, pibuf, rbuf,
              gsem0, gsem1, wsem0, wsem1):
    cid = lax.axis_index("c")
    sid = lax.axis_index("s")
    wid = sid * NC + cid

    pltpu.sync_copy(state_hbm, sbuf)
    pltpu.sync_copy(pos_hbm.at[pl.ds(sid * 2 * PP_PER_TILE, 2 * PP_PER_TILE)], pbuf)

    def build_pair2(pp2, carry):
        for q in range(2):
            ppl = pp2 * 2 + q
            bwsem = wsem0 if q == 0 else wsem1

            def halves(s, c2):
                for d in range(EMBED_DIM // L):
                    lrtmp[s, pl.ds(d * L, L)] = (
                        pbuf[2 * ppl, pl.ds(d * L, L)] + sbuf[s, pl.ds(d * L, L)]
                    )
                    lrtmp[NUM_STATES + s, pl.ds(d * L, L)] = (
                        pbuf[2 * ppl + 1, pl.ds(d * L, L)] + sbuf[s, pl.ds(d * L, L)]
                    )
                return c2

            lax.fori_loop(0, NUM_STATES, halves, 0)

            @pl.when(pp2 >= 1)
            def _wait_prev_slab():
                pltpu.make_async_copy(
                    bbuf.at[q], t_hbm.at[pl.ds(0, NPAIR_STATES)], bwsem
                ).wait()

            def se_body(se, c2):
                lvals = [lrtmp[se, pl.ds(d * L, L)] for d in range(EMBED_DIM // L)]

                def so_body(so, lv):
                    r = se * NUM_STATES + so
                    for d in range(EMBED_DIM // L):
                        bbuf[q, r, pl.ds(d * L, L)] = lv[d]
                        bbuf[q, r, pl.ds(EMBED_DIM + d * L, L)] = (
                            lrtmp[NUM_STATES + so, pl.ds(d * L, L)]
                        )
                    return lv

                lax.fori_loop(0, NUM_STATES, so_body, lvals)
                return c2

            lax.fori_loop(0, NUM_STATES, se_body, 0)
            pltpu.async_copy(
                bbuf.at[q],
                t_hbm.at[pl.ds(
                    cid * T_ROWS + (sid * PP_PER_TILE + ppl) * NPAIR_STATES,
                    NPAIR_STATES)],
                bwsem,
            )
        return carry

    lax.fori_loop(0, PP_PER_TILE // 2, build_pair2, 0)
    for q in range(2):
        bwsem = wsem0 if q == 0 else wsem1
        pltpu.make_async_copy(
            bbuf.at[q], t_hbm.at[pl.ds(0, NPAIR_STATES)], bwsem
        ).wait()

    def pvec_body(k, carry):
        i16 = lax.iota(jnp.int32, L)
        pvec[pl.ds(k * L, L)] = (
            (lax.shift_right_logical(i16, 3) + 2 * k) * NPAIR_STATES
            + cid * T_ROWS
        )
        return carry

    lax.fori_loop(0, (N_PP * GRP) // L, pvec_body, 0)

    plsc.subcore_barrier()

    def group_body(gi, carry):
        g = wid * G_PER_W + gi
        pltpu.sync_copy(grid_hbm.at[pl.ds(g * GRP, GRP)], ibuf)

        def pidx_body(k, c2):
            i16 = lax.iota(jnp.int32, L)
            rvec = lax.bitwise_and(i16, 7)
            cvec = (lax.shift_right_logical(i16, 3) + 2 * k) * 2
            even = plsc.load_gather(ibuf, [rvec, cvec])
            odd = plsc.load_gather(ibuf, [rvec, cvec + 1])
            pibuf[pl.ds(k * L, L)] = (
                even * NUM_STATES + odd + pvec[pl.ds(k * L, L)]
            )
            return c2

        lax.fori_loop(0, (N_PP * GRP) // L, pidx_body, 0)

        for u in range(N_UNITS):
            us = u % 2
            gsem = gsem0 if us == 0 else gsem1
            wsem = wsem0 if us == 0 else wsem1

            if u >= 2:
                pltpu.make_async_copy(
                    rbuf.at[us], out_hbm.at[g, pl.ds(u * UNIT, UNIT)], wsem
                ).wait()
            else:
                @pl.when(gi > 0)
                def _wait_prev_write():
                    pltpu.make_async_copy(
                        rbuf.at[us], out_hbm.at[g, pl.ds(u * UNIT, UNIT)], wsem
                    ).wait()

            gathers = [
                pltpu.async_copy(
                    t_hbm.at[pibuf.at[pl.ds(u * UNIT + j * CHUNK, CHUNK)]],
                    rbuf.at[us, pl.ds(j * CHUNK, CHUNK)],
                    gsem,
                )
                for j in range(UNIT // CHUNK)
            ]
            for gg in gathers:
                gg.wait()
            pltpu.async_copy(
                rbuf.at[us], out_hbm.at[g, pl.ds(u * UNIT, UNIT)], wsem
            )
        return carry

    lax.fori_loop(0, G_PER_W, group_body, 0)

    lastg = wid * G_PER_W + G_PER_W - 1
    for us in range(2):
        wsem = wsem0 if us == 0 else wsem1
        pltpu.make_async_copy(
            rbuf.at[us],
            out_hbm.at[lastg, pl.ds((N_UNITS - 2 + us) * UNIT, UNIT)],
            wsem,
        ).wait()


def kernel(grid_obs, state_table, pos_table):
    out4, _ = _sc_embed(grid_obs.astype(jnp.int32), state_table, pos_table)
    out = (
        out4.reshape(N_GRP, N_PP, GRP, ROW)
        .transpose(0, 2, 1, 3)
        .reshape(BATCH, N_POS * EMBED_DIM)
    )
    return out


# speedup vs baseline: 40.3870x; 1.0349x over previous
import functools

import jax
import jax.numpy as jnp
from jax import lax
from jax.experimental import pallas as pl
from jax.experimental.pallas import tpu as pltpu
from jax.experimental.pallas import tpu_sc as plsc

NUM_STATES = 12
NPAIR_STATES = NUM_STATES * NUM_STATES
EMBED_DIM = 64
ROW = 2 * EMBED_DIM
N_POS = 1024
N_PP = N_POS // 2
BATCH = 4096
GRP = 8
N_GRP = BATCH // GRP

NC = 2
NS = 16
L = 16
NW = NC * NS
G_PER_W = N_GRP // NW
CHUNK = 128
UNIT = 256
N_UNITS = (N_PP * GRP) // UNIT
T_ROWS = N_PP * NPAIR_STATES
PP_PER_TILE = N_PP // NS

_mesh = plsc.VectorSubcoreMesh(core_axis_name="c", subcore_axis_name="s")


@functools.partial(
    pl.kernel,
    out_type=[
        jax.ShapeDtypeStruct((N_GRP, N_PP * GRP, ROW), jnp.float32),
        jax.ShapeDtypeStruct((NC * T_ROWS, ROW), jnp.float32),
    ],
    mesh=_mesh,
    compiler_params=pltpu.CompilerParams(
        use_tc_tiling_on_sc=False, needs_layout_passes=False
    ),
    scratch_types=[
        pltpu.VMEM((NUM_STATES, EMBED_DIM), jnp.float32),
        pltpu.VMEM((2 * PP_PER_TILE, EMBED_DIM), jnp.float32),
        pltpu.VMEM((2, NPAIR_STATES // 2, ROW), jnp.float32),
        pltpu.VMEM((2 * NUM_STATES, EMBED_DIM), jnp.float32),
        pltpu.VMEM((N_PP * GRP,), jnp.int32),
        pltpu.VMEM((2, GRP, N_POS), jnp.int32),
        pltpu.VMEM((2, N_PP * GRP), jnp.int32),
        pltpu.VMEM((2, UNIT, ROW), jnp.float32),
        pltpu.SemaphoreType.DMA,
        pltpu.SemaphoreType.DMA,
        pltpu.SemaphoreType.DMA,
        pltpu.SemaphoreType.DMA,
        pltpu.SemaphoreType.DMA,
    ],
)
def _sc_embed(grid_hbm, state_hbm, pos_hbm, out_hbm, t_hbm,
              sbuf, pbuf, bbuf, lrtmp, pvec, ibuf, pibuf, rbuf,
              gsem0, gsem1, isem, wsem0, wsem1):
    cid = lax.axis_index("c")
    sid = lax.axis_index("s")
    wid = sid * NC + cid

    pltpu.sync_copy(state_hbm, sbuf)
    pltpu.sync_copy(pos_hbm.at[pl.ds(sid * 2 * PP_PER_TILE, 2 * PP_PER_TILE)], pbuf)

    HSLAB = NPAIR_STATES // 2

    def build_pair(ppl, carry):
        def halves(s, c2):
            for d in range(EMBED_DIM // L):
                lrtmp[s, pl.ds(d * L, L)] = (
                    pbuf[2 * ppl, pl.ds(d * L, L)] + sbuf[s, pl.ds(d * L, L)]
                )
                lrtmp[NUM_STATES + s, pl.ds(d * L, L)] = (
                    pbuf[2 * ppl + 1, pl.ds(d * L, L)] + sbuf[s, pl.ds(d * L, L)]
                )
            return c2

        lax.fori_loop(0, NUM_STATES, halves, 0)

        for q in range(2):
            bwsem = wsem0 if q == 0 else wsem1

            @pl.when(ppl >= 1)
            def _wait_prev_slab():
                pltpu.make_async_copy(
                    bbuf.at[q], t_hbm.at[pl.ds(0, HSLAB)], bwsem
                ).wait()

            def se_body(se6, c2):
                se = se6 + q * (NUM_STATES // 2)
                lvals = [lrtmp[se, pl.ds(d * L, L)] for d in range(EMBED_DIM // L)]

                def so_body(so, lv):
                    r = se6 * NUM_STATES + so
                    for d in range(EMBED_DIM // L):
                        bbuf[q, r, pl.ds(d * L, L)] = lv[d]
                        bbuf[q, r, pl.ds(EMBED_DIM + d * L, L)] = (
                            lrtmp[NUM_STATES + so, pl.ds(d * L, L)]
                        )
                    return lv

                lax.fori_loop(0, NUM_STATES, so_body, lvals)
                return c2

            lax.fori_loop(0, NUM_STATES // 2, se_body, 0)
            pltpu.async_copy(
                bbuf.at[q],
                t_hbm.at[pl.ds(
                    cid * T_ROWS + (sid * PP_PER_TILE + ppl) * NPAIR_STATES
                    + q * HSLAB,
                    HSLAB)],
                bwsem,
            )
        return carry

    lax.fori_loop(0, PP_PER_TILE, build_pair, 0)
    for q in range(2):
        bwsem = wsem0 if q == 0 else wsem1
        pltpu.make_async_copy(
            bbuf.at[q], t_hbm.at[pl.ds(0, HSLAB)], bwsem
        ).wait()

    def pvec_body(k, carry):
        i16 = lax.iota(jnp.int32, L)
        pvec[pl.ds(k * L, L)] = (
            (lax.shift_right_logical(i16, 3) + 2 * k) * NPAIR_STATES
            + cid * T_ROWS
        )
        return carry

    lax.fori_loop(0, (N_PP * GRP) // L, pvec_body, 0)

    plsc.subcore_barrier()

    K_PER_UNIT = (N_PP * GRP) // L // N_UNITS

    def make_pidx_chunk(slx, u):
        slv = jnp.full((L,), slx, dtype=jnp.int32)

        def pidx_body(k, c2):
            i16 = lax.iota(jnp.int32, L)
            rvec = lax.bitwise_and(i16, 7)
            cvec = (lax.shift_right_logical(i16, 3) + 2 * k) * 2
            even = plsc.load_gather(ibuf, [slv, rvec, cvec])
            odd = plsc.load_gather(ibuf, [slv, rvec, cvec + 1])
            pibuf[slx, pl.ds(k * L, L)] = (
                even * NUM_STATES + odd + pvec[pl.ds(k * L, L)]
            )
            return c2

        return lax.fori_loop(u * K_PER_UNIT, (u + 1) * K_PER_UNIT, pidx_body, 0)

    g0 = wid * G_PER_W
    pltpu.sync_copy(grid_hbm.at[pl.ds(g0 * GRP, GRP)], ibuf.at[0])
    for u in range(N_UNITS):
        make_pidx_chunk(0, u)
    pltpu.async_copy(grid_hbm.at[pl.ds((g0 + 1) * GRP, GRP)], ibuf.at[1], isem)

    def group_body(gi, carry):
        g = g0 + gi
        sl = lax.rem(gi, 2)

        for u in range(N_UNITS):
            us = u % 2
            gsem = gsem0 if us == 0 else gsem1
            wsem = wsem0 if us == 0 else wsem1

            if u >= 2:
                pltpu.make_async_copy(
                    rbuf.at[us], out_hbm.at[g, pl.ds(u * UNIT, UNIT)], wsem
                ).wait()
            else:
                @pl.when(gi > 0)
                def _wait_prev_write():
                    pltpu.make_async_copy(
                        rbuf.at[us], out_hbm.at[g, pl.ds(u * UNIT, UNIT)], wsem
                    ).wait()

            gathers = [
                pltpu.async_copy(
                    t_hbm.at[pibuf.at[sl, pl.ds(u * UNIT + j * CHUNK, CHUNK)]],
                    rbuf.at[us, pl.ds(j * CHUNK, CHUNK)],
                    gsem,
                )
                for j in range(UNIT // CHUNK)
            ]

            @pl.when(gi < G_PER_W - 1)
            def _shadow_work():
                if u == 0:
                    pltpu.make_async_copy(
                        grid_hbm.at[pl.ds((g + 1) * GRP, GRP)],
                        ibuf.at[1 - sl], isem,
                    ).wait()

                    @pl.when(gi < G_PER_W - 2)
                    def _stage_next_idx():
                        pltpu.async_copy(
                            grid_hbm.at[pl.ds((g + 2) * GRP, GRP)],
                            ibuf.at[sl], isem,
                        )
                make_pidx_chunk(1 - sl, u)

            for gg in gathers:
                gg.wait()
            pltpu.async_copy(
                rbuf.at[us], out_hbm.at[g, pl.ds(u * UNIT, UNIT)], wsem
            )
        return carry

    lax.fori_loop(0, G_PER_W, group_body, 0)

    lastg = wid * G_PER_W + G_PER_W - 1
    for us in range(2):
        wsem = wsem0 if us == 0 else wsem1
        pltpu.make_async_copy(
            rbuf.at[us],
            out_hbm.at[lastg, pl.ds((N_UNITS - 2 + us) * UNIT, UNIT)],
            wsem,
        ).wait()


def kernel(grid_obs, state_table, pos_table):
    out4, _ = _sc_embed(grid_obs.astype(jnp.int32), state_table, pos_table)
    out = (
        out4.reshape(N_GRP, N_PP, GRP, ROW)
        .transpose(0, 2, 1, 3)
        .reshape(BATCH, N_POS * EMBED_DIM)
    )
    return out
